# Initial kernel scaffold; baseline (speedup 1.0000x reference)
import jax
import jax.numpy as jnp
from jax.experimental import pallas as pl


def kernel(x, edge_index, batch_idx, W1_0, b1_0, W2_0, b2_0, gamma_0, beta_0, W1_1, b1_1, W2_1, b2_1, gamma_1, beta_1, W1_2, b1_2, W2_2, b2_2, gamma_2, beta_2, Wp, bp):
    raise NotImplementedError("write your pallas kernel here")


# speedup vs baseline: 4.5829x; 4.5829x over previous
import functools

import jax
import jax.numpy as jnp
from jax import lax
from jax.experimental import pallas as pl
from jax.experimental.pallas import tpu as pltpu
from jax.experimental.pallas import tpu_sc as plsc

N = 10000
N_PAD = 10240
E = 320000
F = 128
G = 64
BN_EPS = 1e-5

NC = 2
NS = 16
NW = NC * NS
E_PER_TILE = E // NW
CHUNK = 80
N_CHUNKS = E_PER_TILE // CHUNK
ROWS_PER_TILE = N_PAD // NS
ZROWS = 128


def _sc_segsum_body(h_hbm, src_hbm, dst_hbm, zero_hbm, out_hbm,
                    src_v, dst_v, rows_v, stage_v, acc, sem):
  c = lax.axis_index("c")
  s = lax.axis_index("s")
  w = c * NS + s

  pltpu.sync_copy(zero_hbm, stage_v)
  row0 = s * ROWS_PER_TILE
  for i in range(ROWS_PER_TILE // ZROWS):
    pltpu.sync_copy(stage_v, acc.at[pl.ds(row0 + i * ZROWS, ZROWS)])
  plsc.subcore_barrier()

  e_base = w * E_PER_TILE

  def step(k, carry):
    base = e_base + k * CHUNK
    pltpu.sync_copy(src_hbm.at[pl.ds(base, CHUNK)], src_v)
    pltpu.sync_copy(dst_hbm.at[pl.ds(base, CHUNK)], dst_v)
    pltpu.async_copy(h_hbm.at[src_v], rows_v, sem).wait()
    pltpu.sync_copy(rows_v, acc.at[dst_v], add=True)
    return carry

  lax.fori_loop(0, N_CHUNKS, step, 0)
  plsc.subcore_barrier()

  for i in range(ROWS_PER_TILE // ZROWS):
    r = row0 + i * ZROWS
    pltpu.sync_copy(acc.at[pl.ds(r, ZROWS)], stage_v)
    pltpu.sync_copy(stage_v, out_hbm.at[c, pl.ds(r, ZROWS)])


_sc_segsum = pl.kernel(
    _sc_segsum_body,
    out_type=jax.ShapeDtypeStruct((NC, N_PAD, F), jnp.float32),
    mesh=plsc.VectorSubcoreMesh(
        core_axis_name="c", subcore_axis_name="s",
        num_cores=NC, num_subcores=NS),
    scratch_types=[
        pltpu.VMEM((CHUNK,), jnp.int32),
        pltpu.VMEM((CHUNK,), jnp.int32),
        pltpu.VMEM((CHUNK, F), jnp.float32),
        pltpu.VMEM((ZROWS, F), jnp.float32),
        pltpu.VMEM_SHARED((N_PAD, F), jnp.float32),
        pltpu.SemaphoreType.DMA,
    ],
)


ROW_BLK = 1024


def _tc_layer_body(h_ref, p_ref, w1_ref, b1_ref, w2_ref, b2_ref,
                   gamma_ref, beta_ref, out_ref):
  a = h_ref[...] + p_ref[0] + p_ref[1]
  t = jnp.maximum(jnp.dot(a, w1_ref[...],
                          preferred_element_type=jnp.float32) + b1_ref[...], 0.0)
  u = jnp.dot(t, w2_ref[...], preferred_element_type=jnp.float32) + b2_ref[...]
  scale = gamma_ref[...] * (1.0 / jnp.sqrt(1.0 + BN_EPS))
  out_ref[...] = jnp.maximum(u * scale + beta_ref[...], 0.0)


def _tc_layer(h, partials, W1, b1, W2, b2, gamma, beta):
  grid = (N_PAD // ROW_BLK,)
  full = pl.BlockSpec((F, F), lambda i: (0, 0))
  vec = pl.BlockSpec((1, F), lambda i: (0, 0))
  return pl.pallas_call(
      _tc_layer_body,
      grid=grid,
      in_specs=[
          pl.BlockSpec((ROW_BLK, F), lambda i: (i, 0)),
          pl.BlockSpec((NC, ROW_BLK, F), lambda i: (0, i, 0)),
          full, vec, full, vec, vec, vec,
      ],
      out_specs=pl.BlockSpec((ROW_BLK, F), lambda i: (i, 0)),
      out_shape=jax.ShapeDtypeStruct((N_PAD, F), jnp.float32),
  )(h, partials, W1, b1.reshape(1, F), W2, b2.reshape(1, F),
    gamma.reshape(1, F), beta.reshape(1, F))


def _tc_pool_body(h_ref, bidx_ref, wp_ref, bp_ref, out_ref):
  groups = lax.broadcasted_iota(jnp.int32, (1, G), 1)
  onehot = jnp.where(bidx_ref[...] == groups, 1.0, 0.0)
  sums = lax.dot_general(onehot, h_ref[...], (((0,), (0,)), ((), ())),
                         preferred_element_type=jnp.float32)
  ones = jnp.ones((N_PAD, 1), dtype=jnp.float32)
  counts = lax.dot_general(onehot, ones, (((0,), (0,)), ((), ())),
                           preferred_element_type=jnp.float32)
  pooled = sums / jnp.maximum(counts, 1.0)
  out = jnp.dot(pooled, wp_ref[...],
                preferred_element_type=jnp.float32) + bp_ref[...]
  out_ref[...] = jnp.maximum(out, 0.0)


def _tc_pool(h, bidx, Wp, bp):
  return pl.pallas_call(
      _tc_pool_body,
      out_shape=jax.ShapeDtypeStruct((G, G), jnp.float32),
  )(h, bidx.reshape(N_PAD, 1), Wp, bp.reshape(1, G))


@jax.jit
def kernel(x, edge_index, batch_idx,
           W1_0, b1_0, W2_0, b2_0, gamma_0, beta_0,
           W1_1, b1_1, W2_1, b2_1, gamma_1, beta_1,
           W1_2, b1_2, W2_2, b2_2, gamma_2, beta_2,
           Wp, bp):
  src = edge_index[0]
  dst = edge_index[1]
  zero_blk = jnp.zeros((ZROWS, F), dtype=jnp.float32)
  h = jnp.pad(x, ((0, N_PAD - N), (0, 0)))
  bidx = jnp.pad(batch_idx, (0, N_PAD - N), constant_values=G)
  layers = [
      (W1_0, b1_0, W2_0, b2_0, gamma_0, beta_0),
      (W1_1, b1_1, W2_1, b2_1, gamma_1, beta_1),
      (W1_2, b1_2, W2_2, b2_2, gamma_2, beta_2),
  ]
  for (W1, b1, W2, b2, g, bt) in layers:
    partials = _sc_segsum(h, src, dst, zero_blk)
    h = _tc_layer(h, partials, W1, b1, W2, b2, g, bt)
  return _tc_pool(h, bidx, Wp, bp)


# speedup vs baseline: 9.4860x; 2.0699x over previous
import functools

import jax
import jax.numpy as jnp
from jax import lax
from jax.experimental import pallas as pl
from jax.experimental.pallas import tpu as pltpu
from jax.experimental.pallas import tpu_sc as plsc

N = 10000
N_PAD = 10240
E = 320000
F = 128
G = 64
BN_EPS = 1e-5

NC = 2
NS = 16
NW = NC * NS
E_PER_TILE = E // NW
CHUNK = 100
N_CHUNKS = E_PER_TILE // CHUNK
ROWS_PER_TILE = N_PAD // NS
ZROWS = 80


def _sc_segsum_body(h_hbm, ei_hbm, zero_hbm, out_hbm,
                    idx0, idx1, rows0, rows1, acc,
                    gsem0, gsem1, ssem0, ssem1):
  c = lax.axis_index("c")
  s = lax.axis_index("s")
  w = c * NS + s
  chunk0 = w * N_CHUNKS

  pltpu.sync_copy(zero_hbm, rows0.at[pl.ds(0, ZROWS)])
  row0 = s * ROWS_PER_TILE
  for i in range(ROWS_PER_TILE // ZROWS):
    pltpu.sync_copy(rows0.at[pl.ds(0, ZROWS)], acc.at[pl.ds(row0 + i * ZROWS, ZROWS)])
  plsc.subcore_barrier()

  pltpu.sync_copy(ei_hbm.at[chunk0], idx0)
  pltpu.async_copy(h_hbm.at[idx0.at[0]], rows0, gsem0)
  pltpu.sync_copy(ei_hbm.at[chunk0 + 1], idx1)
  pltpu.async_copy(h_hbm.at[idx1.at[0]], rows1, gsem1)

  def step2(j, carry):
    k0 = 2 * j
    pltpu.make_async_copy(h_hbm.at[idx0.at[0]], rows0, gsem0).wait()
    pltpu.async_copy(rows0, acc.at[idx0.at[1]], ssem0, add=True)
    pltpu.make_async_copy(rows0, acc.at[idx0.at[1]], ssem0).wait()

    @pl.when(j + 1 < N_CHUNKS // 2)
    def _():
      pltpu.sync_copy(ei_hbm.at[chunk0 + k0 + 2], idx0)
      pltpu.async_copy(h_hbm.at[idx0.at[0]], rows0, gsem0)

    pltpu.make_async_copy(h_hbm.at[idx1.at[0]], rows1, gsem1).wait()
    pltpu.async_copy(rows1, acc.at[idx1.at[1]], ssem1, add=True)
    pltpu.make_async_copy(rows1, acc.at[idx1.at[1]], ssem1).wait()

    @pl.when(j + 1 < N_CHUNKS // 2)
    def _():
      pltpu.sync_copy(ei_hbm.at[chunk0 + k0 + 3], idx1)
      pltpu.async_copy(h_hbm.at[idx1.at[0]], rows1, gsem1)
    return carry

  lax.fori_loop(0, N_CHUNKS // 2, step2, 0)
  plsc.subcore_barrier()

  for i in range(ROWS_PER_TILE // ZROWS):
    r = row0 + i * ZROWS
    pltpu.sync_copy(acc.at[pl.ds(r, ZROWS)], rows0.at[pl.ds(0, ZROWS)])
    pltpu.sync_copy(rows0.at[pl.ds(0, ZROWS)], out_hbm.at[c, pl.ds(r, ZROWS)])


_sc_segsum = pl.kernel(
    _sc_segsum_body,
    out_type=jax.ShapeDtypeStruct((NC, N_PAD, F), jnp.float32),
    mesh=plsc.VectorSubcoreMesh(
        core_axis_name="c", subcore_axis_name="s",
        num_cores=NC, num_subcores=NS),
    scratch_types=[
        pltpu.VMEM((2, CHUNK), jnp.int32),
        pltpu.VMEM((2, CHUNK), jnp.int32),
        pltpu.VMEM((CHUNK, F), jnp.float32),
        pltpu.VMEM((CHUNK, F), jnp.float32),
        pltpu.VMEM_SHARED((N_PAD, F), jnp.float32),
        pltpu.SemaphoreType.DMA,
        pltpu.SemaphoreType.DMA,
        pltpu.SemaphoreType.DMA,
        pltpu.SemaphoreType.DMA,
    ],
)


ROW_BLK = 1024


def _tc_layer_body(h_ref, p_ref, w1_ref, b1_ref, w2_ref, b2_ref,
                   gamma_ref, beta_ref, out_ref):
  a = h_ref[...] + p_ref[0] + p_ref[1]
  t = jnp.maximum(jnp.dot(a, w1_ref[...],
                          preferred_element_type=jnp.float32) + b1_ref[...], 0.0)
  u = jnp.dot(t, w2_ref[...], preferred_element_type=jnp.float32) + b2_ref[...]
  scale = gamma_ref[...] * (1.0 / jnp.sqrt(1.0 + BN_EPS))
  out_ref[...] = jnp.maximum(u * scale + beta_ref[...], 0.0)


def _tc_layer(h, partials, W1, b1, W2, b2, gamma, beta):
  grid = (N_PAD // ROW_BLK,)
  full = pl.BlockSpec((F, F), lambda i: (0, 0))
  vec = pl.BlockSpec((1, F), lambda i: (0, 0))
  return pl.pallas_call(
      _tc_layer_body,
      grid=grid,
      in_specs=[
          pl.BlockSpec((ROW_BLK, F), lambda i: (i, 0)),
          pl.BlockSpec((NC, ROW_BLK, F), lambda i: (0, i, 0)),
          full, vec, full, vec, vec, vec,
      ],
      out_specs=pl.BlockSpec((ROW_BLK, F), lambda i: (i, 0)),
      out_shape=jax.ShapeDtypeStruct((N_PAD, F), jnp.float32),
  )(h, partials, W1, b1.reshape(1, F), W2, b2.reshape(1, F),
    gamma.reshape(1, F), beta.reshape(1, F))


def _tc_pool_body(h_ref, bidx_ref, wp_ref, bp_ref, out_ref):
  groups = lax.broadcasted_iota(jnp.int32, (1, G), 1)
  onehot = jnp.where(bidx_ref[...] == groups, 1.0, 0.0)
  sums = lax.dot_general(onehot, h_ref[...], (((0,), (0,)), ((), ())),
                         preferred_element_type=jnp.float32)
  ones = jnp.ones((N_PAD, 1), dtype=jnp.float32)
  counts = lax.dot_general(onehot, ones, (((0,), (0,)), ((), ())),
                           preferred_element_type=jnp.float32)
  pooled = sums / jnp.maximum(counts, 1.0)
  out = jnp.dot(pooled, wp_ref[...],
                preferred_element_type=jnp.float32) + bp_ref[...]
  out_ref[...] = jnp.maximum(out, 0.0)


def _tc_pool(h, bidx, Wp, bp):
  return pl.pallas_call(
      _tc_pool_body,
      out_shape=jax.ShapeDtypeStruct((G, G), jnp.float32),
  )(h, bidx.reshape(N_PAD, 1), Wp, bp.reshape(1, G))


@jax.jit
def kernel(x, edge_index, batch_idx,
           W1_0, b1_0, W2_0, b2_0, gamma_0, beta_0,
           W1_1, b1_1, W2_1, b2_1, gamma_1, beta_1,
           W1_2, b1_2, W2_2, b2_2, gamma_2, beta_2,
           Wp, bp):
  ei = edge_index.reshape(2, E // CHUNK, CHUNK).transpose(1, 0, 2)
  zero_blk = jnp.zeros((ZROWS, F), dtype=jnp.float32)
  h = jnp.pad(x, ((0, N_PAD - N), (0, 0)))
  bidx = jnp.pad(batch_idx, (0, N_PAD - N), constant_values=G)
  layers = [
      (W1_0, b1_0, W2_0, b2_0, gamma_0, beta_0),
      (W1_1, b1_1, W2_1, b2_1, gamma_1, beta_1),
      (W1_2, b1_2, W2_2, b2_2, gamma_2, beta_2),
  ]
  for (W1, b1, W2, b2, g, bt) in layers:
    partials = _sc_segsum(h, ei, zero_blk)
    h = _tc_layer(h, partials, W1, b1, W2, b2, g, bt)
  return _tc_pool(h, bidx, Wp, bp)
